# Initial kernel scaffold; baseline (speedup 1.0000x reference)
import jax
import jax.numpy as jnp
from jax.experimental import pallas as pl


def kernel(x, b_dec, W_enc, b_enc, W_dec):
    raise NotImplementedError("write your pallas kernel here")


# speedup vs baseline: 9.9582x; 9.9582x over previous
import functools

import jax
import jax.numpy as jnp
from jax.experimental import pallas as pl

BR = 128


def _monotonic_i32(v):
    s = jax.lax.bitcast_convert_type(v, jnp.int32)
    return jnp.where(s < 0, jnp.bitwise_xor(s, jnp.int32(0x7FFFFFFF)), s)


def _encode_select_kernel(x_ref, b_dec_ref, W_enc_ref, b_enc_ref, z_ref, *, k):
    xc = x_ref[...] - b_dec_ref[...]
    w = W_enc_ref[0]
    pre = jax.lax.dot_general(
        xc, w, (((1,), (1,)), ((), ())),
        preferred_element_type=jnp.float32) + b_enc_ref[...]

    mk = _monotonic_i32(pre)

    def body(_, carry):
        lo, hi = carry
        mid = lo + jax.lax.shift_right_logical(hi - lo, 1)
        cnt = jnp.sum((mk >= mid).astype(jnp.int32), axis=1, keepdims=True)
        pred = cnt >= k
        return jnp.where(pred, mid, lo), jnp.where(pred, hi, mid)

    n = mk.shape[0]
    lo0 = jnp.full((n, 1), jnp.iinfo(jnp.int32).min, jnp.int32)
    hi0 = jnp.full((n, 1), jnp.iinfo(jnp.int32).max, jnp.int32)
    thr, _ = jax.lax.fori_loop(0, 32, body, (lo0, hi0))

    z_ref[...] = jnp.where(mk >= thr, jnp.maximum(pre, 0.0), 0.0)


def _decode_loss_kernel(z_ref, W_dec_ref, b_dec_ref, x_ref, xhat_ref, loss_ref):
    zb = z_ref[...]
    w = W_dec_ref[0]
    xh = jax.lax.dot_general(
        zb, w, (((1,), (1,)), ((), ())),
        preferred_element_type=jnp.float32) + b_dec_ref[...]
    xhat_ref[...] = xh
    r = x_ref[...] - xh

    @pl.when((pl.program_id(0) == 0) & (pl.program_id(1) == 0))
    def _():
        loss_ref[...] = jnp.zeros((1, 1), jnp.float32)

    loss_ref[...] += jnp.sum(r * r).reshape(1, 1)


def kernel(x, b_dec, W_enc, b_enc, W_dec):
    B, T, D_IN = x.shape
    D_SAE = W_enc.shape[1]
    K = 64
    nb = B // BR

    x2 = x.reshape(B, T * D_IN)
    b_dec2 = b_dec.reshape(1, T * D_IN)
    b_enc2 = b_enc.reshape(1, T * D_SAE)

    z2 = pl.pallas_call(
        functools.partial(_encode_select_kernel, k=K),
        grid=(T, nb),
        in_specs=[
            pl.BlockSpec((BR, D_IN), lambda t, i: (i, t)),
            pl.BlockSpec((1, D_IN), lambda t, i: (0, t)),
            pl.BlockSpec((1, D_SAE, D_IN), lambda t, i: (t, 0, 0)),
            pl.BlockSpec((1, D_SAE), lambda t, i: (0, t)),
        ],
        out_specs=pl.BlockSpec((BR, D_SAE), lambda t, i: (i, t)),
        out_shape=jax.ShapeDtypeStruct((B, T * D_SAE), jnp.float32),
    )(x2, b_dec2, W_enc, b_enc2)

    xhat2, loss_sum = pl.pallas_call(
        _decode_loss_kernel,
        grid=(T, nb),
        in_specs=[
            pl.BlockSpec((BR, D_SAE), lambda t, i: (i, t)),
            pl.BlockSpec((1, D_IN, D_SAE), lambda t, i: (t, 0, 0)),
            pl.BlockSpec((1, D_IN), lambda t, i: (0, t)),
            pl.BlockSpec((BR, D_IN), lambda t, i: (i, t)),
        ],
        out_specs=[
            pl.BlockSpec((BR, D_IN), lambda t, i: (i, t)),
            pl.BlockSpec((1, 1), lambda t, i: (0, 0)),
        ],
        out_shape=[
            jax.ShapeDtypeStruct((B, T * D_IN), jnp.float32),
            jax.ShapeDtypeStruct((1, 1), jnp.float32),
        ],
    )(z2, W_dec, b_dec2, x2)

    loss = loss_sum[0, 0] / jnp.float32(B * T)
    return (loss, xhat2.reshape(B, T, D_IN), z2.reshape(B, T, D_SAE))


# speedup vs baseline: 11.7612x; 1.1811x over previous
import functools

import jax
import jax.numpy as jnp
from jax.experimental import pallas as pl

BR = 128


def _monotonic_i32(v):
    s = jax.lax.bitcast_convert_type(v, jnp.int32)
    return jnp.where(s < 0, jnp.bitwise_xor(s, jnp.int32(0x7FFFFFFF)), s)


def _search16(v, k):
    n = v.shape[0]
    one = jnp.ones((), jnp.int16)
    zero = jnp.zeros((), jnp.int16)

    def count_ge(t16):
        c = jnp.where(v >= t16, one, zero)
        m = c.shape[1]
        while m > 128 and m % 2 == 0 and (m // 2) % 128 == 0:
            m //= 2
            c = c[:, :m] + c[:, m:]
        if m > 128:
            acc = c[:, :128]
            for j in range(128, m, 128):
                acc = acc + c[:, j:j + 128]
            c = acc
        return jnp.sum(c.astype(jnp.int32), axis=1, keepdims=True)

    def body(_, carry):
        lo, hi = carry
        mid = lo + ((hi - lo) >> 1)
        pred = count_ge(mid.astype(jnp.int16)) >= k
        return jnp.where(pred, mid, lo), jnp.where(pred, hi, mid)

    lo0 = jnp.full((n, 1), -32768, jnp.int32)
    hi0 = jnp.full((n, 1), 32767, jnp.int32)
    ans, _ = jax.lax.fori_loop(0, 16, body, (lo0, hi0))
    return jnp.where(count_ge(jnp.int16(32767)) >= k, jnp.int32(32767), ans)


def _encode_select_kernel(x_ref, b_dec_ref, W_enc_ref, b_enc_ref, z_ref, *, k):
    xc = x_ref[...] - b_dec_ref[...]
    w = W_enc_ref[0]
    pre = jax.lax.dot_general(
        xc, w, (((1,), (1,)), ((), ())),
        preferred_element_type=jnp.float32) + b_enc_ref[...]

    mk = _monotonic_i32(pre)
    k16 = jnp.int16(k)

    hi16 = jax.lax.shift_right_arithmetic(mk, 16).astype(jnp.int16)
    H = _search16(hi16, k16)

    H16 = H.astype(jnp.int16)
    lo16 = jnp.bitwise_xor(mk.astype(jnp.int16), jnp.int16(-0x8000))
    wv = jnp.where(hi16 > H16, jnp.int16(32767),
                   jnp.where(hi16 < H16, jnp.int16(-32768), lo16))
    L = _search16(wv, k16)

    thr = (jax.lax.shift_left(H, 16)
           | (jnp.bitwise_xor(L, jnp.int32(0x8000)) & 0xFFFF))
    z_ref[...] = jnp.where(mk >= thr, jnp.maximum(pre, 0.0), 0.0)


def _decode_loss_kernel(z_ref, W_dec_ref, b_dec_ref, x_ref, xhat_ref, loss_ref):
    zb = z_ref[...]
    w = W_dec_ref[0]
    xh = jax.lax.dot_general(
        zb, w, (((1,), (1,)), ((), ())),
        preferred_element_type=jnp.float32) + b_dec_ref[...]
    xhat_ref[...] = xh
    r = x_ref[...] - xh

    @pl.when((pl.program_id(0) == 0) & (pl.program_id(1) == 0))
    def _():
        loss_ref[...] = jnp.zeros((1, 1), jnp.float32)

    loss_ref[...] += jnp.sum(r * r).reshape(1, 1)


def kernel(x, b_dec, W_enc, b_enc, W_dec):
    B, T, D_IN = x.shape
    D_SAE = W_enc.shape[1]
    K = 64
    nb = B // BR

    x2 = x.reshape(B, T * D_IN)
    b_dec2 = b_dec.reshape(1, T * D_IN)
    b_enc2 = b_enc.reshape(1, T * D_SAE)

    z2 = pl.pallas_call(
        functools.partial(_encode_select_kernel, k=K),
        grid=(T, nb),
        in_specs=[
            pl.BlockSpec((BR, D_IN), lambda t, i: (i, t)),
            pl.BlockSpec((1, D_IN), lambda t, i: (0, t)),
            pl.BlockSpec((1, D_SAE, D_IN), lambda t, i: (t, 0, 0)),
            pl.BlockSpec((1, D_SAE), lambda t, i: (0, t)),
        ],
        out_specs=pl.BlockSpec((BR, D_SAE), lambda t, i: (i, t)),
        out_shape=jax.ShapeDtypeStruct((B, T * D_SAE), jnp.float32),
    )(x2, b_dec2, W_enc, b_enc2)

    xhat2, loss_sum = pl.pallas_call(
        _decode_loss_kernel,
        grid=(T, nb),
        in_specs=[
            pl.BlockSpec((BR, D_SAE), lambda t, i: (i, t)),
            pl.BlockSpec((1, D_IN, D_SAE), lambda t, i: (t, 0, 0)),
            pl.BlockSpec((1, D_IN), lambda t, i: (0, t)),
            pl.BlockSpec((BR, D_IN), lambda t, i: (i, t)),
        ],
        out_specs=[
            pl.BlockSpec((BR, D_IN), lambda t, i: (i, t)),
            pl.BlockSpec((1, 1), lambda t, i: (0, 0)),
        ],
        out_shape=[
            jax.ShapeDtypeStruct((B, T * D_IN), jnp.float32),
            jax.ShapeDtypeStruct((1, 1), jnp.float32),
        ],
    )(z2, W_dec, b_dec2, x2)

    loss = loss_sum[0, 0] / jnp.float32(B * T)
    return (loss, xhat2.reshape(B, T, D_IN), z2.reshape(B, T, D_SAE))
